# Initial kernel scaffold; baseline (speedup 1.0000x reference)
import jax
import jax.numpy as jnp
from jax.experimental import pallas as pl


def kernel(roi_features, batch_bboxes, batch_directions, batch_priorities, ncp_W1, ncp_b1, ncp_W2, ncp_b2, ncp_W3, ncp_b3, nci_W, nci_b, ep_W1, ep_b1, ep_W2, ep_b2, ep_W3, ep_b3, eproc_W1, eproc_b1, eproc_W2, eproc_b2, gnn_root, gnn_bias):
    raise NotImplementedError("write your pallas kernel here")


# speedup vs baseline: 5.3282x; 5.3282x over previous
import numpy as np
import jax
import jax.numpy as jnp
from jax.experimental import pallas as pl
from jax.experimental.pallas import tpu as pltpu

_N = 64
_E = _N * (_N - 1)
_NODE_CH = 1280
_ACT_CH = 4
_BBOX_POS_MAX = 1024.0


def _edge_onehots():
    idx = np.arange(_N)
    ii, jj = np.meshgrid(idx, idx, indexing="ij")
    mask = ii != jj
    src = ii[mask]
    dst = jj[mask]
    S = np.zeros((_E, _N), np.float32)
    S[np.arange(_E), src] = 1.0
    D = np.zeros((_E, _N), np.float32)
    D[np.arange(_E), dst] = 1.0
    return S, D


def _fused(roi, attr, pri,
           W1, b1, W2, b2, W3, b3,
           nciW, ncib,
           epW1s, epW1d, epb1, epW2, epb2, epW3, epb3,
           eprocW1, eprocb1, W2g, b2r,
           root, gbias,
           S, D, Dt,
           out_act, out_nce, out_ea):
    f32 = jnp.float32

    def dot(a, b):
        return jnp.dot(a, b, preferred_element_type=f32,
                       precision=jax.lax.Precision.HIGHEST)

    h = jax.nn.relu(dot(roi[...], W1[...]) + b1[...])
    h = jax.nn.relu(dot(h, W2[...]) + b2[...])
    x = dot(h, W3[...]) + b3[...]
    out_nce[...] = jax.nn.sigmoid(dot(x, nciW[...]) + ncib[...])

    As = dot(attr[...], epW1s[...])
    Ad = dot(attr[...], epW1d[...])
    Sm = S[...]
    Dm = D[...]
    e1 = jax.nn.relu(dot(Sm, As) + dot(Dm, Ad) + epb1[...])
    e2 = jax.nn.relu(dot(e1, epW2[...]) + epb2[...])
    ea3 = jax.nn.sigmoid(dot(e2, epW3[...]) + epb3[...])
    pS = dot(Sm, pri[...])
    pD = dot(Dm, pri[...])
    hp = jnp.where(pS > pD, f32(1.0), f32(0.0))
    ea4 = jnp.concatenate([ea3, hp], axis=1)
    out_ea[...] = ea4

    H = jax.nn.relu(dot(ea4, eprocW1[...]) + eprocb1[...])
    Gall = dot(x, W2g[...])
    cols = []
    for o in range(_ACT_CH):
        Go = Gall[:, o * 128:(o + 1) * 128]
        SGo = dot(Sm, Go)
        To = dot(Dt[...], H * SGo)
        cols.append(jnp.sum(To, axis=1, keepdims=True))
    aggr1 = jnp.concatenate(cols, axis=1)
    bterm = dot(x, b2r[...])
    tot = jnp.sum(bterm, axis=0, keepdims=True)
    aggr = aggr1 + (tot - bterm)
    out_act[...] = aggr + dot(x, root[...]) + gbias[...]


def kernel(roi_features, batch_bboxes, batch_directions, batch_priorities,
           ncp_W1, ncp_b1, ncp_W2, ncp_b2, ncp_W3, ncp_b3,
           nci_W, nci_b,
           ep_W1, ep_b1, ep_W2, ep_b2, ep_W3, ep_b3,
           eproc_W1, eproc_b1, eproc_W2, eproc_b2,
           gnn_root, gnn_bias):
    f32 = jnp.float32
    roi0 = roi_features[0]
    attr0 = jnp.concatenate(
        [batch_bboxes[0] / _BBOX_POS_MAX, batch_directions[0]], axis=-1)
    pri0 = batch_priorities[0][:, None]

    W2g = jnp.transpose(eproc_W2.reshape(128, _NODE_CH, _ACT_CH),
                        (1, 2, 0)).reshape(_NODE_CH, _ACT_CH * 128)
    b2r = eproc_b2.reshape(_NODE_CH, _ACT_CH)
    epW1s = ep_W1[:8]
    epW1d = ep_W1[8:]

    S_np, D_np = _edge_onehots()
    S = jnp.asarray(S_np)
    D = jnp.asarray(D_np)
    Dt = jnp.asarray(D_np.T)

    row = lambda v: v.reshape(1, -1).astype(f32)

    out_shape = (
        jax.ShapeDtypeStruct((_N, _ACT_CH), f32),
        jax.ShapeDtypeStruct((_N, 5), f32),
        jax.ShapeDtypeStruct((_E, 4), f32),
    )
    return pl.pallas_call(
        _fused,
        out_shape=out_shape,
    )(roi0, attr0, pri0,
      ncp_W1, row(ncp_b1), ncp_W2, row(ncp_b2), ncp_W3, row(ncp_b3),
      nci_W, row(nci_b),
      epW1s, epW1d, row(ep_b1), ep_W2, row(ep_b2), ep_W3, row(ep_b3),
      eproc_W1, row(eproc_b1), W2g, b2r,
      gnn_root, row(gnn_bias),
      S, D, Dt)


# speedup vs baseline: 6.6533x; 1.2487x over previous
import numpy as np
import jax
import jax.numpy as jnp
from jax.experimental import pallas as pl
from jax.experimental.pallas import tpu as pltpu

_N = 64
_E = _N * (_N - 1)
_NODE_CH = 1280
_ACT_CH = 4
_BBOX_POS_MAX = 1024.0


def _edge_onehots():
    idx = np.arange(_N)
    ii, jj = np.meshgrid(idx, idx, indexing="ij")
    mask = ii != jj
    src = ii[mask]
    dst = jj[mask]
    S = np.zeros((_E, _N), np.float32)
    S[np.arange(_E), src] = 1.0
    D = np.zeros((_E, _N), np.float32)
    D[np.arange(_E), dst] = 1.0
    return S, D


def _fused(roi, attr, pri2,
           W1, b1, W2, b2, W3, b3,
           nciW, ncib,
           epW1sd, epb1, epW2, epb2, epW3, epb3,
           eprocW1, eprocb1, W2g, b2r,
           root, gbias,
           S, SD, Dt, Rsum,
           out_act, out_nce, out_ea):
    f32 = jnp.float32

    def dot(a, b, prec=jax.lax.Precision.HIGHEST):
        return jnp.dot(a, b, preferred_element_type=f32, precision=prec)

    h = jax.nn.relu(dot(roi[...], W1[...]) + b1[...])
    h = jax.nn.relu(dot(h, W2[...]) + b2[...])
    x = dot(h, W3[...]) + b3[...]
    out_nce[...] = jax.nn.sigmoid(dot(x, nciW[...]) + ncib[...])

    AsAd = dot(attr[...], epW1sd[...])
    AB = jnp.concatenate([AsAd[:, :256], AsAd[:, 256:]], axis=0)
    SDm = SD[...]
    e1 = jax.nn.relu(dot(SDm, AB) + epb1[...])
    e2 = jax.nn.relu(dot(e1, epW2[...]) + epb2[...])
    ea3 = jax.nn.sigmoid(dot(e2, epW3[...]) + epb3[...])
    pdiff = dot(SDm, pri2[...], prec=jax.lax.Precision.HIGHEST)
    hp = jnp.where(pdiff > 0, f32(1.0), f32(0.0))
    ea4 = jnp.concatenate([ea3, hp], axis=1)
    out_ea[...] = ea4

    H = jax.nn.relu(dot(ea4, eprocW1[...]) + eprocb1[...])
    Gall = dot(x, W2g[...])
    SG = dot(S[...], Gall)
    Ht = jnp.concatenate([H, H, H, H], axis=1)
    T = dot(Dt[...], Ht * SG)
    aggr1 = dot(T, Rsum[...])
    bterm = dot(x, b2r[...])
    tot = jnp.sum(bterm, axis=0, keepdims=True)
    aggr = aggr1 + (tot - bterm)
    out_act[...] = aggr + dot(x, root[...]) + gbias[...]


def kernel(roi_features, batch_bboxes, batch_directions, batch_priorities,
           ncp_W1, ncp_b1, ncp_W2, ncp_b2, ncp_W3, ncp_b3,
           nci_W, nci_b,
           ep_W1, ep_b1, ep_W2, ep_b2, ep_W3, ep_b3,
           eproc_W1, eproc_b1, eproc_W2, eproc_b2,
           gnn_root, gnn_bias):
    f32 = jnp.float32
    roi0 = roi_features[0]
    attr0 = jnp.concatenate(
        [batch_bboxes[0] / _BBOX_POS_MAX, batch_directions[0]], axis=-1)
    pri0 = batch_priorities[0][:, None]

    pri2 = jnp.concatenate([pri0, -pri0], axis=0)

    W2g = jnp.transpose(eproc_W2.reshape(128, _NODE_CH, _ACT_CH),
                        (1, 2, 0)).reshape(_NODE_CH, _ACT_CH * 128)
    b2r = eproc_b2.reshape(_NODE_CH, _ACT_CH)
    epW1sd = jnp.concatenate([ep_W1[:8], ep_W1[8:]], axis=1)

    S_np, D_np = _edge_onehots()
    S = jnp.asarray(S_np)
    SD = jnp.asarray(np.concatenate([S_np, D_np], axis=1))
    Dt = jnp.asarray(D_np.T)
    Rsum_np = np.zeros((_ACT_CH * 128, _ACT_CH), np.float32)
    for o in range(_ACT_CH):
        Rsum_np[o * 128:(o + 1) * 128, o] = 1.0
    Rsum = jnp.asarray(Rsum_np)

    row = lambda v: v.reshape(1, -1).astype(f32)

    out_shape = (
        jax.ShapeDtypeStruct((_N, _ACT_CH), f32),
        jax.ShapeDtypeStruct((_N, 5), f32),
        jax.ShapeDtypeStruct((_E, 4), f32),
    )
    return pl.pallas_call(
        _fused,
        out_shape=out_shape,
    )(roi0, attr0, pri2,
      ncp_W1, row(ncp_b1), ncp_W2, row(ncp_b2), ncp_W3, row(ncp_b3),
      nci_W, row(nci_b),
      epW1sd, row(ep_b1), ep_W2, row(ep_b2), ep_W3, row(ep_b3),
      eproc_W1, row(eproc_b1), W2g, b2r,
      gnn_root, row(gnn_bias),
      S, SD, Dt, Rsum)


# speedup vs baseline: 9.5802x; 1.4399x over previous
import numpy as np
import jax
import jax.numpy as jnp
from jax.experimental import pallas as pl
from jax.experimental.pallas import tpu as pltpu

_N = 64
_E = _N * (_N - 1)
_NODE_CH = 1280
_ACT_CH = 4
_BBOX_POS_MAX = 1024.0


def _edge_onehots():
    idx = np.arange(_N)
    ii, jj = np.meshgrid(idx, idx, indexing="ij")
    mask = ii != jj
    src = ii[mask]
    dst = jj[mask]
    S = np.zeros((_E, _N), np.float32)
    S[np.arange(_E), src] = 1.0
    D = np.zeros((_E, _N), np.float32)
    D[np.arange(_E), dst] = 1.0
    return S, D


def _fused(roi, attr, pri2,
           W1, b1, W2, b2, W3, b3,
           nciW, ncib,
           epW1sd, epb1, epW2, epb2, epW3, epb3,
           eprocW1, eprocb1, W2g, b2r,
           root, gbias,
           S, SD, Dt, Rsum,
           out_act, out_nce, out_ea):
    f32 = jnp.float32
    bf16 = jnp.bfloat16

    def _split(a):
        hi = a.astype(bf16)
        lo = (a - hi.astype(f32)).astype(bf16)
        return hi, lo

    def _d(u, v):
        return jnp.dot(u, v, preferred_element_type=f32)

    def dot(a, b):
        ahi, alo = _split(a)
        bhi, blo = _split(b)
        return _d(ahi, bhi) + _d(ahi, blo) + _d(alo, bhi)

    def odot(s, b):
        sb = s.astype(bf16)
        bhi, blo = _split(b)
        return _d(sb, bhi) + _d(sb, blo)

    def odot_r(a, s):
        sb = s.astype(bf16)
        ahi, alo = _split(a)
        return _d(ahi, sb) + _d(alo, sb)

    h = jax.nn.relu(dot(roi[...], W1[...]) + b1[...])
    h = jax.nn.relu(dot(h, W2[...]) + b2[...])
    x = dot(h, W3[...]) + b3[...]
    out_nce[...] = jax.nn.sigmoid(dot(x, nciW[...]) + ncib[...])

    AsAd = dot(attr[...], epW1sd[...])
    AB = jnp.concatenate([AsAd[:, :256], AsAd[:, 256:]], axis=0)
    SDm = SD[...]
    e1 = jax.nn.relu(odot(SDm, AB) + epb1[...])
    e2 = jax.nn.relu(dot(e1, epW2[...]) + epb2[...])
    ea3 = jax.nn.sigmoid(dot(e2, epW3[...]) + epb3[...])
    pdiff = jnp.dot(SDm, pri2[...], preferred_element_type=f32,
                    precision=jax.lax.Precision.HIGHEST)
    hp = jnp.where(pdiff > 0, f32(1.0), f32(0.0))
    ea4 = jnp.concatenate([ea3, hp], axis=1)
    out_ea[...] = ea4

    H = jax.nn.relu(dot(ea4, eprocW1[...]) + eprocb1[...])
    Gall = dot(x, W2g[...])
    SG = odot(S[...], Gall)
    Ht = jnp.concatenate([H, H, H, H], axis=1)
    T = odot(Dt[...], Ht * SG)
    aggr1 = odot_r(T, Rsum[...])
    bterm = dot(x, b2r[...])
    tot = jnp.sum(bterm, axis=0, keepdims=True)
    aggr = aggr1 + (tot - bterm)
    out_act[...] = aggr + dot(x, root[...]) + gbias[...]


def kernel(roi_features, batch_bboxes, batch_directions, batch_priorities,
           ncp_W1, ncp_b1, ncp_W2, ncp_b2, ncp_W3, ncp_b3,
           nci_W, nci_b,
           ep_W1, ep_b1, ep_W2, ep_b2, ep_W3, ep_b3,
           eproc_W1, eproc_b1, eproc_W2, eproc_b2,
           gnn_root, gnn_bias):
    f32 = jnp.float32
    roi0 = roi_features[0]
    attr0 = jnp.concatenate(
        [batch_bboxes[0] / _BBOX_POS_MAX, batch_directions[0]], axis=-1)
    pri0 = batch_priorities[0][:, None]

    pri2 = jnp.concatenate([pri0, -pri0], axis=0)

    W2g = jnp.transpose(eproc_W2.reshape(128, _NODE_CH, _ACT_CH),
                        (1, 2, 0)).reshape(_NODE_CH, _ACT_CH * 128)
    b2r = eproc_b2.reshape(_NODE_CH, _ACT_CH)
    epW1sd = jnp.concatenate([ep_W1[:8], ep_W1[8:]], axis=1)

    S_np, D_np = _edge_onehots()
    S = jnp.asarray(S_np)
    SD = jnp.asarray(np.concatenate([S_np, D_np], axis=1))
    Dt = jnp.asarray(D_np.T)
    Rsum_np = np.zeros((_ACT_CH * 128, _ACT_CH), np.float32)
    for o in range(_ACT_CH):
        Rsum_np[o * 128:(o + 1) * 128, o] = 1.0
    Rsum = jnp.asarray(Rsum_np)

    row = lambda v: v.reshape(1, -1).astype(f32)

    out_shape = (
        jax.ShapeDtypeStruct((_N, _ACT_CH), f32),
        jax.ShapeDtypeStruct((_N, 5), f32),
        jax.ShapeDtypeStruct((_E, 4), f32),
    )
    return pl.pallas_call(
        _fused,
        out_shape=out_shape,
    )(roi0, attr0, pri2,
      ncp_W1, row(ncp_b1), ncp_W2, row(ncp_b2), ncp_W3, row(ncp_b3),
      nci_W, row(nci_b),
      epW1sd, row(ep_b1), ep_W2, row(ep_b2), ep_W3, row(ep_b3),
      eproc_W1, row(eproc_b1), W2g, b2r,
      gnn_root, row(gnn_bias),
      S, SD, Dt, Rsum)


# speedup vs baseline: 10.3633x; 1.0817x over previous
import numpy as np
import jax
import jax.numpy as jnp
from jax.experimental import pallas as pl
from jax.experimental.pallas import tpu as pltpu

_N = 64
_E = _N * (_N - 1)
_NODE_CH = 1280
_ACT_CH = 4
_BBOX_POS_MAX = 1024.0


def _edge_onehots():
    idx = np.arange(_N)
    ii, jj = np.meshgrid(idx, idx, indexing="ij")
    mask = ii != jj
    src = ii[mask]
    dst = jj[mask]
    S = np.zeros((_E, _N), np.float32)
    S[np.arange(_E), src] = 1.0
    D = np.zeros((_E, _N), np.float32)
    D[np.arange(_E), dst] = 1.0
    return S, D


def _fused(roi, attr, pri_col, pri_row,
           W1, b1, W2, b2, W3, b3,
           ncib,
           epW1sd, epb1, epW2, epb2, epW3, epb3,
           eprocW1, eprocb1, Wx,
           gbias,
           S, SD, D, Dt, Rsum,
           out_act, out_nce, out_ea):
    f32 = jnp.float32
    bf16 = jnp.bfloat16

    def _split(a):
        hi = a.astype(bf16)
        lo = (a - hi.astype(f32)).astype(bf16)
        return hi, lo

    def _d(u, v):
        return jnp.dot(u, v, preferred_element_type=f32)

    def dot(a, b):
        ahi, alo = _split(a)
        bhi, blo = _split(b)
        return _d(ahi, bhi) + _d(ahi, blo) + _d(alo, bhi)

    def odot(sb, b):
        bhi, blo = _split(b)
        return _d(sb, bhi) + _d(sb, blo)

    def odot_r(a, sb):
        ahi, alo = _split(a)
        return _d(ahi, sb) + _d(alo, sb)

    h = jax.nn.relu(dot(roi[...], W1[...]) + b1[...])
    h = jax.nn.relu(dot(h, W2[...]) + b2[...])
    x = dot(h, W3[...]) + b3[...]

    XC = dot(x, Wx[...])
    Gall = XC[:, 0:512]
    out_nce[...] = jax.nn.sigmoid(XC[:, 512:517] + ncib[...])
    rootterm = XC[:, 517:521]
    bterm = XC[:, 521:525]

    AsAd = dot(attr[...], epW1sd[...])
    AB = jnp.concatenate([AsAd[:, :256], AsAd[:, 256:]], axis=0)
    e1 = jax.nn.relu(odot(SD[...], AB) + epb1[...])
    e2 = jax.nn.relu(dot(e1, epW2[...]) + epb2[...])
    ea3 = jax.nn.sigmoid(dot(e2, epW3[...]) + epb3[...])
    pdd = pri_col[...] - pri_row[...]
    Pd = jnp.where(pdd > 0.0, f32(1.0), f32(0.0)).astype(bf16)
    SPd = _d(S[...], Pd)
    hp = jnp.sum(SPd * D[...].astype(f32), axis=1, keepdims=True)
    ea4 = jnp.concatenate([ea3, hp], axis=1)
    out_ea[...] = ea4

    H = jax.nn.relu(dot(ea4, eprocW1[...]) + eprocb1[...])
    SG = odot(S[...], Gall)
    Ht = jnp.concatenate([H, H, H, H], axis=1)
    T = odot(Dt[...], Ht * SG)
    aggr1 = odot_r(T, Rsum[...])
    tot = jnp.sum(bterm, axis=0, keepdims=True)
    aggr = aggr1 + (tot - bterm)
    out_act[...] = aggr + rootterm + gbias[...]


def kernel(roi_features, batch_bboxes, batch_directions, batch_priorities,
           ncp_W1, ncp_b1, ncp_W2, ncp_b2, ncp_W3, ncp_b3,
           nci_W, nci_b,
           ep_W1, ep_b1, ep_W2, ep_b2, ep_W3, ep_b3,
           eproc_W1, eproc_b1, eproc_W2, eproc_b2,
           gnn_root, gnn_bias):
    f32 = jnp.float32
    roi0 = roi_features[0]
    attr0 = jnp.concatenate(
        [batch_bboxes[0] / _BBOX_POS_MAX, batch_directions[0]], axis=-1)
    pri0 = batch_priorities[0][:, None]

    pri_row = batch_priorities[0][None, :]

    W2g = jnp.transpose(eproc_W2.reshape(128, _NODE_CH, _ACT_CH),
                        (1, 2, 0)).reshape(_NODE_CH, _ACT_CH * 128)
    b2r = eproc_b2.reshape(_NODE_CH, _ACT_CH)
    Wx = jnp.concatenate([W2g, nci_W, gnn_root, b2r], axis=1)
    epW1sd = jnp.concatenate([ep_W1[:8], ep_W1[8:]], axis=1)

    S_np, D_np = _edge_onehots()
    bf = jnp.bfloat16
    S = jnp.asarray(S_np, dtype=bf)
    SD = jnp.asarray(np.concatenate([S_np, D_np], axis=1), dtype=bf)
    D = jnp.asarray(D_np, dtype=bf)
    Dt = jnp.asarray(D_np.T, dtype=bf)
    Rsum_np = np.zeros((_ACT_CH * 128, _ACT_CH), np.float32)
    for o in range(_ACT_CH):
        Rsum_np[o * 128:(o + 1) * 128, o] = 1.0
    Rsum = jnp.asarray(Rsum_np, dtype=bf)

    row = lambda v: v.reshape(1, -1).astype(f32)

    out_shape = (
        jax.ShapeDtypeStruct((_N, _ACT_CH), f32),
        jax.ShapeDtypeStruct((_N, 5), f32),
        jax.ShapeDtypeStruct((_E, 4), f32),
    )
    return pl.pallas_call(
        _fused,
        out_shape=out_shape,
    )(roi0, attr0, pri0, pri_row,
      ncp_W1, row(ncp_b1), ncp_W2, row(ncp_b2), ncp_W3, row(ncp_b3),
      row(nci_b),
      epW1sd, row(ep_b1), ep_W2, row(ep_b2), ep_W3, row(ep_b3),
      eproc_W1, row(eproc_b1), Wx,
      row(gnn_bias),
      S, SD, D, Dt, Rsum)
